# Initial kernel scaffold; baseline (speedup 1.0000x reference)
import jax
import jax.numpy as jnp
from jax.experimental import pallas as pl


def kernel(translations, image_indices, point_indices, is_calibrated, cam_translations, points_3d, scales):
    raise NotImplementedError("write your pallas kernel here")


# speedup vs baseline: 2.9945x; 2.9945x over previous
import jax
import jax.numpy as jnp
from jax import lax
from jax.experimental import pallas as pl
from jax.experimental.pallas import tpu as pltpu
from jax.experimental.pallas import tpu_sc as plsc

NUM_IMGS = 5000
NUM_PTS = 500000
NUM_OBS = 2000000

NC = 2
NS = 16
NW = NC * NS
L = 16

C = 2048
NFULL = NUM_OBS // C
TAIL = NUM_OBS - NFULL * C
TAIL_WORKER = NFULL % NW
MAXJ = -(-NFULL // NW)


def _rsqrt_nr(s2):
    bits = plsc.bitcast(s2, jnp.int32)
    bits = jnp.int32(0x5F3759DF) - lax.shift_right_arithmetic(bits, 1)
    y = plsc.bitcast(bits, jnp.float32)
    half = jnp.float32(0.5) * s2
    for _ in range(3):
        y = y * (jnp.float32(1.5) - half * y * y)
    return y


def _body(trans_hbm, iidx_hbm, pidx_hbm, calib_hbm, cam_hbm, pts_hbm,
          scale_hbm, out_hbm,
          iidx_v, pidx_v, trans_v, scale_v, pts_v, out_v, cam_v, calib_v,
          sem):
    wid = lax.axis_index("s") * NC + lax.axis_index("c")

    pltpu.sync_copy(cam_hbm, cam_v)
    pltpu.sync_copy(calib_hbm, calib_v)

    iota = lax.iota(jnp.int32, L)
    iota3 = iota * 3
    col0 = jnp.zeros((L,), jnp.int32)
    col1 = jnp.full((L,), 1, jnp.int32)
    col2 = jnp.full((L,), 2, jnp.int32)
    one_i = jnp.full((L,), 1, jnp.int32)
    two_i = jnp.full((L,), 2, jnp.int32)

    def do_chunk(off, n):
        pltpu.sync_copy(iidx_hbm.at[pl.ds(off, n)], iidx_v.at[pl.ds(0, n)])
        pltpu.sync_copy(pidx_hbm.at[pl.ds(off, n)], pidx_v.at[pl.ds(0, n)])
        pltpu.sync_copy(trans_hbm.at[pl.ds(3 * off, 3 * n)],
                        trans_v.at[pl.ds(0, 3 * n)])
        pltpu.sync_copy(scale_hbm.at[pl.ds(off, n)], scale_v.at[pl.ds(0, n)])
        pltpu.async_copy(pts_hbm.at[pidx_v.at[pl.ds(0, n)]],
                         pts_v.at[pl.ds(0, n)], sem).wait()

        def group(g, carry):
            o = g * L
            rows = o + iota
            r3 = 3 * o + iota3
            img = plsc.load_gather(iidx_v, [rows])
            i3 = img * 3
            tx = plsc.load_gather(trans_v, [r3])
            ty = plsc.load_gather(trans_v, [r3 + one_i])
            tz = plsc.load_gather(trans_v, [r3 + two_i])
            px = plsc.load_gather(pts_v, [rows, col0])
            py = plsc.load_gather(pts_v, [rows, col1])
            pz = plsc.load_gather(pts_v, [rows, col2])
            cx = plsc.load_gather(cam_v, [i3])
            cy = plsc.load_gather(cam_v, [i3 + one_i])
            cz = plsc.load_gather(cam_v, [i3 + two_i])
            cal = plsc.load_gather(calib_v, [img])
            s = plsc.load_gather(scale_v, [rows])

            s2 = tx * tx + ty * ty + tz * tz
            rinv = jnp.where(s2 >= jnp.float32(1e-16), _rsqrt_nr(s2),
                             jnp.float32(1e8))
            f = s * jnp.where(cal > jnp.float32(0.5), jnp.float32(1.0), rinv)
            plsc.store_scatter(out_v, [r3], px - cx - f * tx)
            plsc.store_scatter(out_v, [r3 + one_i], py - cy - f * ty)
            plsc.store_scatter(out_v, [r3 + two_i], pz - cz - f * tz)
            return carry

        lax.fori_loop(0, n // L, group, 0, unroll=2)
        pltpu.sync_copy(out_v.at[pl.ds(0, 3 * n)],
                        out_hbm.at[pl.ds(3 * off, 3 * n)])

    nj = (NFULL - wid + NW - 1) // NW

    def chunk_step(j, carry):
        chunk = wid + NW * j
        do_chunk(pl.multiple_of(chunk * C, C), C)
        return carry

    lax.fori_loop(0, nj, chunk_step, 0)

    if TAIL:
        @pl.when(wid == TAIL_WORKER)
        def _():
            do_chunk(NFULL * C, TAIL)


@jax.jit
def _run(trans_flat, image_indices, point_indices, calib_f,
         cam_flat, points_3d, scales_flat):
    mesh = plsc.VectorSubcoreMesh(core_axis_name="c", subcore_axis_name="s")
    kfn = pl.kernel(
        _body,
        out_type=jax.ShapeDtypeStruct((3 * NUM_OBS,), jnp.float32),
        mesh=mesh,
        compiler_params=pltpu.CompilerParams(needs_layout_passes=False,
                                             use_tc_tiling_on_sc=False),
        scratch_types=[
            pltpu.VMEM((C,), jnp.int32),
            pltpu.VMEM((C,), jnp.int32),
            pltpu.VMEM((3 * C,), jnp.float32),
            pltpu.VMEM((C,), jnp.float32),
            pltpu.VMEM((C, 8), jnp.float32),
            pltpu.VMEM((3 * C,), jnp.float32),
            pltpu.VMEM((3 * NUM_IMGS,), jnp.float32),
            pltpu.VMEM((NUM_IMGS,), jnp.float32),
            pltpu.SemaphoreType.DMA,
        ],
    )
    return kfn(trans_flat, image_indices, point_indices, calib_f,
               cam_flat, points_3d, scales_flat)


def kernel(translations, image_indices, point_indices, is_calibrated,
           cam_translations, points_3d, scales):
    points8 = jnp.pad(points_3d, ((0, 0), (0, 5)))
    out = _run(translations.reshape(-1),
               image_indices.astype(jnp.int32),
               point_indices.astype(jnp.int32),
               is_calibrated.astype(jnp.float32),
               cam_translations.reshape(-1),
               points8,
               scales.reshape(-1))
    return out.reshape(NUM_OBS, 3)


# speedup vs baseline: 48.2243x; 16.1043x over previous
import jax
import jax.numpy as jnp
from jax import lax
from jax.experimental import pallas as pl
from jax.experimental.pallas import tpu as pltpu
from jax.experimental.pallas import tpu_sc as plsc

NUM_IMGS = 5000
NUM_PTS = 500000
NUM_OBS = 2000000

NC = 2
NS = 16
NW = NC * NS
L = 16

C = 4096
NFULL = NUM_OBS // C
TAIL = NUM_OBS - NFULL * C
TAIL_WORKER = NFULL % NW


def _rsqrt_nr(s2):
    bits = plsc.bitcast(s2, jnp.int32)
    bits = jnp.int32(0x5F3759DF) - lax.shift_right_arithmetic(bits, 1)
    y = plsc.bitcast(bits, jnp.float32)
    half = jnp.float32(0.5) * s2
    for _ in range(3):
        y = y * (jnp.float32(1.5) - half * y * y)
    return y


def _body(tx_hbm, ty_hbm, tz_hbm, iidx_hbm, pidx_hbm, calib_hbm,
          cx_hbm, cy_hbm, cz_hbm, px_hbm, py_hbm, pz_hbm, s_hbm,
          ox_hbm, oy_hbm, oz_hbm,
          iidx_v, pidx_v, tx_v, ty_v, tz_v, s_v, gx_v, gy_v, gz_v,
          ox_v, oy_v, oz_v, cam_v, calib_v,
          sem_idx, sem_in, sem_g, sem_out):
    wid = lax.axis_index("s") * NC + lax.axis_index("c")

    pltpu.sync_copy(cx_hbm, cam_v.at[0])
    pltpu.sync_copy(cy_hbm, cam_v.at[1])
    pltpu.sync_copy(cz_hbm, cam_v.at[2])
    pltpu.sync_copy(calib_hbm, calib_v)

    dsl = pl.ds(0, C)

    def fire_in(off, b):
        ssl = pl.ds(off, C)
        pltpu.async_copy(pidx_hbm.at[ssl], pidx_v.at[b, dsl], sem_idx.at[b])
        pltpu.async_copy(iidx_hbm.at[ssl], iidx_v.at[b, dsl], sem_in.at[b])
        pltpu.async_copy(tx_hbm.at[ssl], tx_v.at[b, dsl], sem_in.at[b])
        pltpu.async_copy(ty_hbm.at[ssl], ty_v.at[b, dsl], sem_in.at[b])
        pltpu.async_copy(tz_hbm.at[ssl], tz_v.at[b, dsl], sem_in.at[b])
        pltpu.async_copy(s_hbm.at[ssl], s_v.at[b, dsl], sem_in.at[b])
        pltpu.make_async_copy(pidx_hbm.at[ssl], pidx_v.at[b, dsl],
                              sem_idx.at[b]).wait()
        idx = pidx_v.at[b, dsl]
        pltpu.async_copy(px_hbm.at[idx], gx_v.at[b, dsl], sem_g.at[b])
        pltpu.async_copy(py_hbm.at[idx], gy_v.at[b, dsl], sem_g.at[b])
        pltpu.async_copy(pz_hbm.at[idx], gz_v.at[b, dsl], sem_g.at[b])

    def wait_in(b):
        for ref, hbm in ((iidx_v, iidx_hbm), (tx_v, tx_hbm), (ty_v, ty_hbm),
                         (tz_v, tz_hbm), (s_v, s_hbm)):
            pltpu.make_async_copy(hbm.at[pl.ds(0, C)],
                                  ref.at[b, dsl], sem_in.at[b]).wait()
        for ref in (gx_v, gy_v, gz_v):
            pltpu.make_async_copy(px_hbm.at[pl.ds(0, C)],
                                  ref.at[b, dsl], sem_g.at[b]).wait()

    def drain_out(b, n):
        nsl = pl.ds(0, n)
        pltpu.make_async_copy(ox_v.at[b, nsl], ox_hbm.at[nsl],
                              sem_out.at[b]).wait()
        pltpu.make_async_copy(oy_v.at[b, nsl], oy_hbm.at[nsl],
                              sem_out.at[b]).wait()
        pltpu.make_async_copy(oz_v.at[b, nsl], oz_hbm.at[nsl],
                              sem_out.at[b]).wait()

    def compute(b, n):
        def group(g, carry):
            o = g * L
            sl = pl.ds(o, L)
            img = iidx_v[b, sl]
            tx = tx_v[b, sl]
            ty = ty_v[b, sl]
            tz = tz_v[b, sl]
            px = gx_v[b, sl]
            py = gy_v[b, sl]
            pz = gz_v[b, sl]
            s = s_v[b, sl]
            cx = plsc.load_gather(cam_v, [jnp.zeros((L,), jnp.int32), img])
            cy = plsc.load_gather(cam_v, [jnp.full((L,), 1, jnp.int32), img])
            cz = plsc.load_gather(cam_v, [jnp.full((L,), 2, jnp.int32), img])
            cal = plsc.load_gather(calib_v, [img])

            s2 = tx * tx + ty * ty + tz * tz
            rinv = jnp.where(s2 >= jnp.float32(1e-16), _rsqrt_nr(s2),
                             jnp.float32(1e8))
            f = s * jnp.where(cal > jnp.float32(0.5), jnp.float32(1.0), rinv)
            ox_v[b, sl] = px - cx - f * tx
            oy_v[b, sl] = py - cy - f * ty
            oz_v[b, sl] = pz - cz - f * tz
            return carry

        lax.fori_loop(0, n // L, group, 0, unroll=4)

    def fire_out(off, b, n):
        nsl = pl.ds(0, n)
        ssl = pl.ds(off, n)
        pltpu.async_copy(ox_v.at[b, nsl], ox_hbm.at[ssl], sem_out.at[b])
        pltpu.async_copy(oy_v.at[b, nsl], oy_hbm.at[ssl], sem_out.at[b])
        pltpu.async_copy(oz_v.at[b, nsl], oz_hbm.at[ssl], sem_out.at[b])

    nj = (NFULL - wid + NW - 1) // NW

    def step(j, carry):
        b = j % 2
        bb = 1 - b

        @pl.when(j < nj)
        def _():
            chunk = wid + NW * j
            fire_in(pl.multiple_of(chunk * C, C), b)

        @pl.when(j >= 1)
        def _():
            wait_in(bb)

            @pl.when(j >= 3)
            def _():
                drain_out(bb, C)

            compute(bb, C)
            chunkp = wid + NW * (j - 1)
            fire_out(pl.multiple_of(chunkp * C, C), bb, C)

        return carry

    lax.fori_loop(0, nj + 1, step, 0)
    drain_out(0, C)
    drain_out(1, C)

    if TAIL:
        @pl.when(wid == TAIL_WORKER)
        def _():
            off = NFULL * C
            nsl = pl.ds(0, TAIL)
            ssl = pl.ds(off, TAIL)
            pltpu.async_copy(pidx_hbm.at[ssl], pidx_v.at[0, nsl],
                             sem_idx.at[0])
            pltpu.async_copy(iidx_hbm.at[ssl], iidx_v.at[0, nsl],
                             sem_in.at[0])
            pltpu.async_copy(tx_hbm.at[ssl], tx_v.at[0, nsl], sem_in.at[0])
            pltpu.async_copy(ty_hbm.at[ssl], ty_v.at[0, nsl], sem_in.at[0])
            pltpu.async_copy(tz_hbm.at[ssl], tz_v.at[0, nsl], sem_in.at[0])
            pltpu.async_copy(s_hbm.at[ssl], s_v.at[0, nsl], sem_in.at[0])
            pltpu.make_async_copy(pidx_hbm.at[ssl], pidx_v.at[0, nsl],
                                  sem_idx.at[0]).wait()
            idx = pidx_v.at[0, nsl]
            pltpu.async_copy(px_hbm.at[idx], gx_v.at[0, nsl], sem_g.at[0])
            pltpu.async_copy(py_hbm.at[idx], gy_v.at[0, nsl], sem_g.at[0])
            pltpu.async_copy(pz_hbm.at[idx], gz_v.at[0, nsl], sem_g.at[0])
            for ref, hbm in ((iidx_v, iidx_hbm), (tx_v, tx_hbm),
                             (ty_v, ty_hbm), (tz_v, tz_hbm), (s_v, s_hbm)):
                pltpu.make_async_copy(hbm.at[pl.ds(0, TAIL)],
                                      ref.at[0, nsl], sem_in.at[0]).wait()
            for ref in (gx_v, gy_v, gz_v):
                pltpu.make_async_copy(px_hbm.at[pl.ds(0, TAIL)],
                                      ref.at[0, nsl], sem_g.at[0]).wait()
            compute(0, TAIL)
            pltpu.sync_copy(ox_v.at[0, nsl], ox_hbm.at[ssl])
            pltpu.sync_copy(oy_v.at[0, nsl], oy_hbm.at[ssl])
            pltpu.sync_copy(oz_v.at[0, nsl], oz_hbm.at[ssl])


@jax.jit
def _run(tx, ty, tz, iidx, pidx, calib_f, cx, cy, cz, px, py, pz, s):
    mesh = plsc.VectorSubcoreMesh(core_axis_name="c", subcore_axis_name="s")
    kfn = pl.kernel(
        _body,
        out_type=(jax.ShapeDtypeStruct((NUM_OBS,), jnp.float32),
                  jax.ShapeDtypeStruct((NUM_OBS,), jnp.float32),
                  jax.ShapeDtypeStruct((NUM_OBS,), jnp.float32)),
        mesh=mesh,
        compiler_params=pltpu.CompilerParams(needs_layout_passes=False,
                                             use_tc_tiling_on_sc=False,
                                             disable_bounds_checks=True),
        scratch_types=[
            pltpu.VMEM((2, C), jnp.int32),
            pltpu.VMEM((2, C), jnp.int32),
            pltpu.VMEM((2, C), jnp.float32),
            pltpu.VMEM((2, C), jnp.float32),
            pltpu.VMEM((2, C), jnp.float32),
            pltpu.VMEM((2, C), jnp.float32),
            pltpu.VMEM((2, C), jnp.float32),
            pltpu.VMEM((2, C), jnp.float32),
            pltpu.VMEM((2, C), jnp.float32),
            pltpu.VMEM((2, C), jnp.float32),
            pltpu.VMEM((2, C), jnp.float32),
            pltpu.VMEM((2, C), jnp.float32),
            pltpu.VMEM((3, NUM_IMGS), jnp.float32),
            pltpu.VMEM((NUM_IMGS,), jnp.float32),
            pltpu.SemaphoreType.DMA((2,)),
            pltpu.SemaphoreType.DMA((2,)),
            pltpu.SemaphoreType.DMA((2,)),
            pltpu.SemaphoreType.DMA((2,)),
        ],
    )
    return kfn(tx, ty, tz, iidx, pidx, calib_f, cx, cy, cz, px, py, pz, s)


def kernel(translations, image_indices, point_indices, is_calibrated,
           cam_translations, points_3d, scales):
    ox, oy, oz = _run(
        translations[:, 0], translations[:, 1], translations[:, 2],
        image_indices.astype(jnp.int32), point_indices.astype(jnp.int32),
        is_calibrated.astype(jnp.float32),
        cam_translations[:, 0], cam_translations[:, 1],
        cam_translations[:, 2],
        points_3d[:, 0], points_3d[:, 1], points_3d[:, 2],
        scales.reshape(-1))
    return jnp.stack([ox, oy, oz], axis=1)
